# baseline (speedup 1.0000x reference)
import numpy as np
import jax
import jax.numpy as jnp
from jax import lax
from jax.experimental import pallas as pl
from jax.experimental.pallas import tpu as pltpu
from jax.experimental.pallas import tpu_sc as plsc

H = 512
W = 512
C = 96
HW = H * W

NC = 2
NS = 16
NW = NC * NS
PPW = HW // NW
P = 128
NCHUNK = PPW // P
L = 16
CW = C // 2
WG = CW // L

_PERM = np.empty((C,), dtype=np.int32)
for _b in range(WG):
    for _t in range(L):
        _PERM[32 * _b + 2 * _t] = 32 * _b + _t
        _PERM[32 * _b + 2 * _t + 1] = 32 * _b + 16 + _t


def _bcast_lane(v, j):
    idx = jnp.full((L,), j, dtype=jnp.int32)
    return lax.gather(
        v, idx[:, None],
        lax.GatherDimensionNumbers(
            offset_dims=(), collapsed_slice_dims=(0,), start_index_map=(0,)),
        slice_sizes=(1,),
        mode=lax.GatherScatterMode.PROMISE_IN_BOUNDS)


_HIMASK = -65536


def _unpack2(wword):
    lo = plsc.bitcast(wword << 16, jnp.float32)
    hi = plsc.bitcast(wword & _HIMASK, jnp.float32)
    return lo, hi


def _sc_grid_sample(tab, gxyp):
    mesh = plsc.VectorSubcoreMesh(core_axis_name="c", subcore_axis_name="s")

    def body(tab_hbm, gxyp_hbm, out_hbm, stage, idxs, ws, rs, outs, gsems, osems):
        wid = lax.axis_index("s") * NC + lax.axis_index("c")
        base = wid * PPW

        def fire(ci, s):
            pltpu.sync_copy(gxyp_hbm.at[pl.ds((base + ci * P) * 3, 3 * P)], stage)
            idxv = idxs[s]
            wv = ws[s]
            for g in range(P // L):
                o = g * L
                gx16 = stage[pl.ds(o, L)]
                gy16 = stage[pl.ds(P + o, L)]
                pad16 = stage[pl.ds(2 * P + o, L)]
                ix = ((gx16 + 1.0) * W - 1.0) / 2.0
                iy = ((gy16 + 1.0) * H - 1.0) / 2.0
                tx = ix.astype(jnp.int32)
                ty = iy.astype(jnp.int32)
                x0 = jnp.where(ix < tx.astype(jnp.float32), tx - 1, tx)
                y0 = jnp.where(iy < ty.astype(jnp.float32), ty - 1, ty)
                wx1 = ix - x0.astype(jnp.float32)
                wy1 = iy - y0.astype(jnp.float32)
                wx0 = 1.0 - wx1
                wy0 = 1.0 - wy1
                x1 = x0 + 1
                y1 = y0 + 1
                vx0 = (x0 >= 0) & (x0 <= W - 1)
                vx1 = (x1 >= 0) & (x1 <= W - 1)
                vy0 = (y0 >= 0) & (y0 <= H - 1)
                vy1 = (y1 >= 0) & (y1 <= H - 1)
                zero = jnp.zeros((L,), jnp.float32)
                w00 = jnp.where(vy0 & vx0, wy0 * wx0, zero)
                w01 = jnp.where(vy0 & vx1, wy0 * wx1, zero)
                w10 = jnp.where(vy1 & vx0, wy1 * wx0, zero)
                w11 = jnp.where(vy1 & vx1, wy1 * wx1, zero)
                m = w00 + w01 + w10 + w11
                x0c = jnp.clip(x0, 0, W - 1)
                x1c = jnp.clip(x1, 0, W - 1)
                yb0 = jnp.clip(y0, 0, H - 1) * W
                yb1 = jnp.clip(y1, 0, H - 1) * W
                idxv[0, pl.ds(o, L)] = yb0 + x0c
                idxv[1, pl.ds(o, L)] = yb0 + x1c
                idxv[2, pl.ds(o, L)] = yb1 + x0c
                idxv[3, pl.ds(o, L)] = yb1 + x1c
                wv[0, pl.ds(o, L)] = w00 * m
                wv[1, pl.ds(o, L)] = w01 * m
                wv[2, pl.ds(o, L)] = w10 * m
                wv[3, pl.ds(o, L)] = w11 * m
                wv[4, pl.ds(o, L)] = pad16 * (1.0 - m)
            for k in range(4):
                pltpu.async_copy(tab_hbm.at[idxv.at[k]], rs[s][k], gsems[s])

        def drain_combine(ci, s):
            off = base + ci * P
            for k in range(4):
                pltpu.make_async_copy(
                    tab_hbm.at[idxs[s].at[k]], rs[s][k], gsems[s]).wait()
            @pl.when(ci >= 2)
            def _():
                pltpu.make_async_copy(
                    outs[s], out_hbm.at[pl.ds(off - 2 * P, P)], osems[s]).wait()
            r0, r1, r2, r3 = rs[s]
            wv = ws[s]
            outv = outs[s]

            def comb(g, _):
                o = g * L
                w00g = wv[0, pl.ds(o, L)]
                w01g = wv[1, pl.ds(o, L)]
                w10g = wv[2, pl.ds(o, L)]
                w11g = wv[3, pl.ds(o, L)]
                ptg = wv[4, pl.ds(o, L)]
                for j in range(L):
                    p = o + j
                    b00 = _bcast_lane(w00g, j)
                    b01 = _bcast_lane(w01g, j)
                    b10 = _bcast_lane(w10g, j)
                    b11 = _bcast_lane(w11g, j)
                    bpt = _bcast_lane(ptg, j)
                    for wg in range(WG):
                        sw = wg * L
                        lo0, hi0 = _unpack2(r0[p, pl.ds(sw, L)])
                        lo1, hi1 = _unpack2(r1[p, pl.ds(sw, L)])
                        lo2, hi2 = _unpack2(r2[p, pl.ds(sw, L)])
                        lo3, hi3 = _unpack2(r3[p, pl.ds(sw, L)])
                        acc_lo = b00 * lo0 + bpt
                        acc_lo = acc_lo + b01 * lo1
                        acc_lo = acc_lo + b10 * lo2
                        acc_lo = acc_lo + b11 * lo3
                        acc_hi = b00 * hi0 + bpt
                        acc_hi = acc_hi + b01 * hi1
                        acc_hi = acc_hi + b10 * hi2
                        acc_hi = acc_hi + b11 * hi3
                        outv[p, pl.ds(32 * wg, L)] = acc_lo
                        outv[p, pl.ds(32 * wg + 16, L)] = acc_hi
                return 0

            lax.fori_loop(0, P // L, comb, 0)
            pltpu.async_copy(outv, out_hbm.at[pl.ds(off, P)], osems[s])

        fire(0, 0)

        def body2(k2, _):
            ci = k2 * 2

            @pl.when(ci + 1 < NCHUNK)
            def _():
                fire(ci + 1, 1)

            drain_combine(ci, 0)

            @pl.when(ci + 2 < NCHUNK)
            def _():
                fire(ci + 2, 0)

            @pl.when(ci + 1 < NCHUNK)
            def _():
                drain_combine(ci + 1, 1)

            return 0

        lax.fori_loop(0, (NCHUNK + 1) // 2, body2, 0)

        pltpu.make_async_copy(
            outs[0], out_hbm.at[pl.ds(base + (NCHUNK - 2) * P, P)], osems[0]).wait()
        pltpu.make_async_copy(
            outs[1], out_hbm.at[pl.ds(base + (NCHUNK - 1) * P, P)], osems[1]).wait()

    f = pl.kernel(
        body,
        out_type=jax.ShapeDtypeStruct((HW, C), jnp.float32),
        mesh=mesh,
        scratch_types=[
            pltpu.VMEM((3 * P,), jnp.float32),
            [pltpu.VMEM((4, P), jnp.int32) for _ in range(2)],
            [pltpu.VMEM((5, P), jnp.float32) for _ in range(2)],
            [[pltpu.VMEM((P, CW), jnp.int32) for _ in range(4)]
             for _ in range(2)],
            [pltpu.VMEM((P, C), jnp.float32) for _ in range(2)],
            [pltpu.SemaphoreType.DMA for _ in range(2)],
            [pltpu.SemaphoreType.DMA for _ in range(2)],
        ],
        compiler_params=pltpu.CompilerParams(
            use_tc_tiling_on_sc=False, needs_layout_passes=False),
    )
    return f(tab, gxyp)


def kernel(input, grid, input_mask, padding_buf):
    inp_perm = input[0].reshape(C, HW)[_PERM]
    t = inp_perm.astype(jnp.bfloat16).T
    tab = jax.lax.bitcast_convert_type(
        t.reshape(HW, CW, 2), jnp.int32)
    gx = grid[0, :, :, 0].reshape(-1, P)
    gy = grid[0, :, :, 1].reshape(-1, P)
    pad = padding_buf[0, 0].reshape(-1, P)
    gxyp = jnp.stack([gx, gy, pad], axis=1).reshape(-1)
    out_t = _sc_grid_sample(tab, gxyp)
    return out_t.T.reshape(1, C, H, W)


# speedup vs baseline: 1.2245x; 1.2245x over previous
import jax
import jax.numpy as jnp
from jax import lax
from jax.experimental import pallas as pl
from jax.experimental.pallas import tpu as pltpu
from jax.experimental.pallas import tpu_sc as plsc

H = 512
W = 512
C = 96
HW = H * W

NC = 2
NS = 16
NW = NC * NS
PPW = HW // NW
P = 128
NCHUNK = PPW // P
L = 16
WG = C // 32

_HIMASK = -65536


def _bcast_lane(v, j):
    idx = jnp.full((L,), j, dtype=jnp.int32)
    return lax.gather(
        v, idx[:, None],
        lax.GatherDimensionNumbers(
            offset_dims=(), collapsed_slice_dims=(0,), start_index_map=(0,)),
        slice_sizes=(1,),
        mode=lax.GatherScatterMode.PROMISE_IN_BOUNDS)


def _unpack2(row32):
    wword = plsc.bitcast(row32, jnp.int32)
    even = plsc.bitcast(wword << 16, jnp.float32)
    odd = plsc.bitcast(wword & _HIMASK, jnp.float32)
    return even, odd


def _sc_grid_sample(tab, gxy, pad):
    mesh = plsc.VectorSubcoreMesh(core_axis_name="c", subcore_axis_name="s")

    def body(tab_hbm, gxy_hbm, pad_hbm, out_hbm,
             gxyv, padv, idxs, ws, rs, outs, gsems, osems):
        wid = lax.axis_index("s") * NC + lax.axis_index("c")
        base = wid * PPW
        pltpu.sync_copy(gxy_hbm.at[pl.ds(base * 2, PPW * 2)], gxyv)
        pltpu.sync_copy(pad_hbm.at[pl.ds(base, PPW)], padv)
        lane2 = lax.iota(jnp.int32, L) * 2

        def fire(ci, s):
            idxv = idxs[s]
            wv = ws[s]
            for g in range(P // L):
                o = g * L
                gbase = ci * (2 * P) + 2 * o
                gx16 = plsc.load_gather(gxyv, [lane2 + gbase])
                gy16 = plsc.load_gather(gxyv, [lane2 + (gbase + 1)])
                pad16 = padv[pl.ds(ci * P + o, L)]
                ix = ((gx16 + 1.0) * W - 1.0) / 2.0
                iy = ((gy16 + 1.0) * H - 1.0) / 2.0
                tx = ix.astype(jnp.int32)
                ty = iy.astype(jnp.int32)
                x0 = jnp.where(ix < tx.astype(jnp.float32), tx - 1, tx)
                y0 = jnp.where(iy < ty.astype(jnp.float32), ty - 1, ty)
                wx1 = ix - x0.astype(jnp.float32)
                wy1 = iy - y0.astype(jnp.float32)
                wx0 = 1.0 - wx1
                wy0 = 1.0 - wy1
                x1 = x0 + 1
                y1 = y0 + 1
                vx0 = (x0 >= 0) & (x0 <= W - 1)
                vx1 = (x1 >= 0) & (x1 <= W - 1)
                vy0 = (y0 >= 0) & (y0 <= H - 1)
                vy1 = (y1 >= 0) & (y1 <= H - 1)
                zero = jnp.zeros((L,), jnp.float32)
                w00 = jnp.where(vy0 & vx0, wy0 * wx0, zero)
                w01 = jnp.where(vy0 & vx1, wy0 * wx1, zero)
                w10 = jnp.where(vy1 & vx0, wy1 * wx0, zero)
                w11 = jnp.where(vy1 & vx1, wy1 * wx1, zero)
                m = w00 + w01 + w10 + w11
                x0c = jnp.clip(x0, 0, W - 1)
                x1c = jnp.clip(x1, 0, W - 1)
                yb0 = jnp.clip(y0, 0, H - 1) * W
                yb1 = jnp.clip(y1, 0, H - 1) * W
                idxv[0, pl.ds(o, L)] = yb0 + x0c
                idxv[1, pl.ds(o, L)] = yb0 + x1c
                idxv[2, pl.ds(o, L)] = yb1 + x0c
                idxv[3, pl.ds(o, L)] = yb1 + x1c
                wv[0, pl.ds(o, L)] = w00 * m
                wv[1, pl.ds(o, L)] = w01 * m
                wv[2, pl.ds(o, L)] = w10 * m
                wv[3, pl.ds(o, L)] = w11 * m
                wv[4, pl.ds(o, L)] = pad16 * (1.0 - m)
            for k in range(4):
                pltpu.async_copy(tab_hbm.at[idxv.at[k]], rs[s][k], gsems[s])

        def drain_combine(ci, s):
            off = base + ci * P
            for k in range(4):
                pltpu.make_async_copy(
                    tab_hbm.at[idxs[s].at[k]], rs[s][k], gsems[s]).wait()
            @pl.when(ci >= 2)
            def _():
                pltpu.make_async_copy(
                    outs[s], out_hbm.at[pl.ds(off - 2 * P, P)], osems[s]).wait()
            r0, r1, r2, r3 = rs[s]
            wv = ws[s]
            outv = outs[s]

            def comb(g, _):
                o = g * L
                w00g = wv[0, pl.ds(o, L)]
                w01g = wv[1, pl.ds(o, L)]
                w10g = wv[2, pl.ds(o, L)]
                w11g = wv[3, pl.ds(o, L)]
                ptg = wv[4, pl.ds(o, L)]
                for j in range(L):
                    p = o + j
                    b00 = _bcast_lane(w00g, j)
                    b01 = _bcast_lane(w01g, j)
                    b10 = _bcast_lane(w10g, j)
                    b11 = _bcast_lane(w11g, j)
                    bpt = _bcast_lane(ptg, j)
                    for wg in range(WG):
                        sw = wg * 32
                        e0, o0 = _unpack2(r0[p, pl.ds(sw, 32)])
                        e1, o1 = _unpack2(r1[p, pl.ds(sw, 32)])
                        e2, o2 = _unpack2(r2[p, pl.ds(sw, 32)])
                        e3, o3 = _unpack2(r3[p, pl.ds(sw, 32)])
                        acc_e = b00 * e0 + bpt
                        acc_e = acc_e + b01 * e1
                        acc_e = acc_e + b10 * e2
                        acc_e = acc_e + b11 * e3
                        acc_o = b00 * o0 + bpt
                        acc_o = acc_o + b01 * o1
                        acc_o = acc_o + b10 * o2
                        acc_o = acc_o + b11 * o3
                        outv[p, pl.ds(sw, 32)] = plsc.pack(
                            acc_e, acc_o, format=plsc.PackFormat.INTERLEAVED)
                return 0

            lax.fori_loop(0, P // L, comb, 0)
            pltpu.async_copy(outv, out_hbm.at[pl.ds(off, P)], osems[s])

        fire(0, 0)

        def body2(k2, _):
            ci = k2 * 2

            @pl.when(ci + 1 < NCHUNK)
            def _():
                fire(ci + 1, 1)

            drain_combine(ci, 0)

            @pl.when(ci + 2 < NCHUNK)
            def _():
                fire(ci + 2, 0)

            @pl.when(ci + 1 < NCHUNK)
            def _():
                drain_combine(ci + 1, 1)

            return 0

        lax.fori_loop(0, (NCHUNK + 1) // 2, body2, 0)

        pltpu.make_async_copy(
            outs[0], out_hbm.at[pl.ds(base + (NCHUNK - 2) * P, P)], osems[0]).wait()
        pltpu.make_async_copy(
            outs[1], out_hbm.at[pl.ds(base + (NCHUNK - 1) * P, P)], osems[1]).wait()

    f = pl.kernel(
        body,
        out_type=jax.ShapeDtypeStruct((HW, C), jnp.bfloat16),
        mesh=mesh,
        scratch_types=[
            pltpu.VMEM((PPW * 2,), jnp.float32),
            pltpu.VMEM((PPW,), jnp.float32),
            [pltpu.VMEM((4, P), jnp.int32) for _ in range(2)],
            [pltpu.VMEM((5, P), jnp.float32) for _ in range(2)],
            [[pltpu.VMEM((P, C), jnp.bfloat16) for _ in range(4)]
             for _ in range(2)],
            [pltpu.VMEM((P, C), jnp.bfloat16) for _ in range(2)],
            [pltpu.SemaphoreType.DMA for _ in range(2)],
            [pltpu.SemaphoreType.DMA for _ in range(2)],
        ],
        compiler_params=pltpu.CompilerParams(
            use_tc_tiling_on_sc=False, needs_layout_passes=False),
    )
    return f(tab, gxy, pad)


def kernel(input, grid, input_mask, padding_buf):
    tab = input[0].reshape(C, HW).astype(jnp.bfloat16).T
    gxy = grid.reshape(HW * 2)
    pad = padding_buf.reshape(HW)
    out_t = _sc_grid_sample(tab, gxy, pad)
    return out_t.T.astype(jnp.float32).reshape(1, C, H, W)


# speedup vs baseline: 1.8579x; 1.5174x over previous
import jax
import jax.numpy as jnp
from jax import lax
from jax.experimental import pallas as pl
from jax.experimental.pallas import tpu as pltpu
from jax.experimental.pallas import tpu_sc as plsc

H = 512
W = 512
C = 96
CP = 128
HW = H * W

NC = 2
NS = 16
NW = NC * NS
PPW = HW // NW
P = 64
NCHUNK = PPW // P
L = 16
CB = C // L


def _bcast_lane(v, j):
    idx = jnp.full((L,), j, dtype=jnp.int32)
    return lax.gather(
        v, idx[:, None],
        lax.GatherDimensionNumbers(
            offset_dims=(), collapsed_slice_dims=(0,), start_index_map=(0,)),
        slice_sizes=(1,),
        mode=lax.GatherScatterMode.PROMISE_IN_BOUNDS)


def _sc_grid_sample(tab, gxy, pad):
    mesh = plsc.VectorSubcoreMesh(core_axis_name="c", subcore_axis_name="s")

    def body(tab_hbm, gxy_hbm, pad_hbm, out_hbm,
             gxyv, padv, idxs, ws, rs, outs, gsems, osems):
        wid = lax.axis_index("s") * NC + lax.axis_index("c")
        base = wid * PPW
        pltpu.sync_copy(gxy_hbm.at[pl.ds(base * 2, PPW * 2)], gxyv)
        pltpu.sync_copy(pad_hbm.at[pl.ds(base, PPW)], padv)
        lane2 = lax.iota(jnp.int32, L) * 2

        def fire(ci, s):
            idxv = idxs[s]
            wv = ws[s]
            for g in range(P // L):
                o = g * L
                gbase = ci * (2 * P) + 2 * o
                gx16 = plsc.load_gather(gxyv, [lane2 + gbase])
                gy16 = plsc.load_gather(gxyv, [lane2 + (gbase + 1)])
                pad16 = padv[pl.ds(ci * P + o, L)]
                ix = ((gx16 + 1.0) * W - 1.0) / 2.0
                iy = ((gy16 + 1.0) * H - 1.0) / 2.0
                tx = ix.astype(jnp.int32)
                ty = iy.astype(jnp.int32)
                x0 = jnp.where(ix < tx.astype(jnp.float32), tx - 1, tx)
                y0 = jnp.where(iy < ty.astype(jnp.float32), ty - 1, ty)
                wx1 = ix - x0.astype(jnp.float32)
                wy1 = iy - y0.astype(jnp.float32)
                wx0 = 1.0 - wx1
                wy0 = 1.0 - wy1
                x1 = x0 + 1
                y1 = y0 + 1
                vx0 = (x0 >= 0) & (x0 <= W - 1)
                vx1 = (x1 >= 0) & (x1 <= W - 1)
                vy0 = (y0 >= 0) & (y0 <= H - 1)
                vy1 = (y1 >= 0) & (y1 <= H - 1)
                zero = jnp.zeros((L,), jnp.float32)
                w00 = jnp.where(vy0 & vx0, wy0 * wx0, zero)
                w01 = jnp.where(vy0 & vx1, wy0 * wx1, zero)
                w10 = jnp.where(vy1 & vx0, wy1 * wx0, zero)
                w11 = jnp.where(vy1 & vx1, wy1 * wx1, zero)
                m = w00 + w01 + w10 + w11
                x0c = jnp.clip(x0, 0, W - 1)
                x1c = jnp.clip(x1, 0, W - 1)
                yb0 = jnp.clip(y0, 0, H - 1) * W
                yb1 = jnp.clip(y1, 0, H - 1) * W
                idxv[0, pl.ds(o, L)] = yb0 + x0c
                idxv[1, pl.ds(o, L)] = yb0 + x1c
                idxv[2, pl.ds(o, L)] = yb1 + x0c
                idxv[3, pl.ds(o, L)] = yb1 + x1c
                wv[0, pl.ds(o, L)] = w00 * m
                wv[1, pl.ds(o, L)] = w01 * m
                wv[2, pl.ds(o, L)] = w10 * m
                wv[3, pl.ds(o, L)] = w11 * m
                wv[4, pl.ds(o, L)] = pad16 * (1.0 - m)
            for k in range(4):
                pltpu.async_copy(tab_hbm.at[idxv.at[k]], rs[s][k], gsems[s])

        def drain_combine(ci, s):
            off = base + ci * P
            for k in range(4):
                pltpu.make_async_copy(
                    tab_hbm.at[idxs[s].at[k]], rs[s][k], gsems[s]).wait()
            @pl.when(ci >= 2)
            def _():
                pltpu.make_async_copy(
                    outs[s], out_hbm.at[pl.ds(off - 2 * P, P)], osems[s]).wait()
            r0, r1, r2, r3 = rs[s]
            wv = ws[s]
            outv = outs[s]

            def comb(g, _):
                o = g * L
                w00g = wv[0, pl.ds(o, L)]
                w01g = wv[1, pl.ds(o, L)]
                w10g = wv[2, pl.ds(o, L)]
                w11g = wv[3, pl.ds(o, L)]
                ptg = wv[4, pl.ds(o, L)]
                for j in range(L):
                    p = o + j
                    b00 = _bcast_lane(w00g, j)
                    b01 = _bcast_lane(w01g, j)
                    b10 = _bcast_lane(w10g, j)
                    b11 = _bcast_lane(w11g, j)
                    bpt = _bcast_lane(ptg, j)
                    for cb in range(CB):
                        cs = cb * L
                        acc = b00 * r0[p, pl.ds(cs, L)] + bpt
                        acc = acc + b01 * r1[p, pl.ds(cs, L)]
                        acc = acc + b10 * r2[p, pl.ds(cs, L)]
                        acc = acc + b11 * r3[p, pl.ds(cs, L)]
                        outv[p, pl.ds(cs, L)] = acc
                return 0

            lax.fori_loop(0, P // L, comb, 0)
            pltpu.async_copy(outv, out_hbm.at[pl.ds(off, P)], osems[s])

        fire(0, 0)

        def body2(k2, _):
            ci = k2 * 2

            @pl.when(ci + 1 < NCHUNK)
            def _():
                fire(ci + 1, 1)

            drain_combine(ci, 0)

            @pl.when(ci + 2 < NCHUNK)
            def _():
                fire(ci + 2, 0)

            @pl.when(ci + 1 < NCHUNK)
            def _():
                drain_combine(ci + 1, 1)

            return 0

        lax.fori_loop(0, (NCHUNK + 1) // 2, body2, 0)

        pltpu.make_async_copy(
            outs[0], out_hbm.at[pl.ds(base + (NCHUNK - 2) * P, P)], osems[0]).wait()
        pltpu.make_async_copy(
            outs[1], out_hbm.at[pl.ds(base + (NCHUNK - 1) * P, P)], osems[1]).wait()

    f = pl.kernel(
        body,
        out_type=jax.ShapeDtypeStruct((HW, CP), jnp.float32),
        mesh=mesh,
        scratch_types=[
            pltpu.VMEM((PPW * 2,), jnp.float32),
            pltpu.VMEM((PPW,), jnp.float32),
            [pltpu.VMEM((4, P), jnp.int32) for _ in range(2)],
            [pltpu.VMEM((5, P), jnp.float32) for _ in range(2)],
            [[pltpu.VMEM((P, CP), jnp.float32) for _ in range(4)]
             for _ in range(2)],
            [pltpu.VMEM((P, CP), jnp.float32) for _ in range(2)],
            [pltpu.SemaphoreType.DMA for _ in range(2)],
            [pltpu.SemaphoreType.DMA for _ in range(2)],
        ],
        compiler_params=pltpu.CompilerParams(needs_layout_passes=False),
    )
    return f(tab, gxy, pad)


def kernel(input, grid, input_mask, padding_buf):
    tab = jnp.pad(input[0].reshape(C, HW), ((0, CP - C), (0, 0))).T
    gxy = grid.reshape(HW * 2)
    pad = padding_buf.reshape(HW)
    out_t = _sc_grid_sample(tab, gxy, pad)
    return out_t[:, :C].T.reshape(1, C, H, W)
